# baseline (speedup 1.0000x reference)
import functools

import jax
import jax.numpy as jnp
from jax import lax
from jax.experimental import pallas as pl
from jax.experimental.pallas import tpu as pltpu
from jax.experimental.pallas import tpu_sc as plsc

N_NODES = 10000
N_EDGES = 320000
F = 128
EPS = 1e-6

NC = 2
NS = 16
NW = NC * NS
EDGES_PER_TILE = N_EDGES // NW
CHUNK = 128
NCHUNKS = -(-EDGES_PER_TILE // CHUNK)
TILE_E = NCHUNKS * CHUNK
N_PAD = 10112
ROWS_PER_TILE = N_PAD // NS

ROW_BLOCK = 2000


def _segment_sum_sc(support, src2d, dst3d):
    mesh = plsc.VectorSubcoreMesh(core_axis_name="c", subcore_axis_name="s")

    @functools.partial(
        pl.kernel,
        out_type=jax.ShapeDtypeStruct((NC, N_PAD, F), jnp.float32),
        mesh=mesh,
        scratch_types=[
            pltpu.VMEM((TILE_E,), jnp.int32),
            pltpu.VMEM((CHUNK,), jnp.int32),
            pltpu.VMEM((CHUNK,), jnp.int32),
            pltpu.VMEM((CHUNK, F), jnp.float32),
            pltpu.VMEM((CHUNK, F), jnp.float32),
            pltpu.VMEM_SHARED((N_PAD, F), jnp.float32),
            pltpu.SemaphoreType.DMA,
            pltpu.SemaphoreType.DMA,
            pltpu.SemaphoreType.DMA,
            pltpu.SemaphoreType.DMA,
            pltpu.SemaphoreType.DMA,
            pltpu.SemaphoreType.DMA,
        ],
    )
    def kern(sup_hbm, src_hbm, dst_hbm, out_hbm,
             src_v, dv0, dv1, buf0, buf1, acc, g0, g1, s0, s1, i0, i1):
        cid = lax.axis_index("c")
        sid = lax.axis_index("s")
        wid = cid * NS + sid

        pltpu.sync_copy(src_hbm.at[wid], src_v)

        @pl.loop(0, CHUNK)
        def _zr(r):
            @pl.loop(0, F, step=16)
            def _zc(c):
                buf0[r, pl.ds(c, 16)] = jnp.zeros((16,), jnp.float32)

        base = sid * ROWS_PER_TILE

        @pl.loop(0, ROWS_PER_TILE - CHUNK, step=CHUNK)
        def _za(r0):
            pltpu.sync_copy(buf0, acc.at[pl.ds(base + r0, CHUNK)])

        pltpu.sync_copy(
            buf0.at[pl.ds(0, ROWS_PER_TILE % CHUNK)],
            acc.at[pl.ds(base + ROWS_PER_TILE - ROWS_PER_TILE % CHUNK,
                         ROWS_PER_TILE % CHUNK)],
        )

        plsc.subcore_barrier()

        def start_idx(c, dv, sem):
            pltpu.make_async_copy(dst_hbm.at[wid].at[c], dv, sem).start()

        def wait_idx(dv, sem):
            pltpu.make_async_copy(dst_hbm.at[wid].at[0], dv, sem).wait()

        def start_gather(c, buf, sem):
            pltpu.make_async_copy(
                sup_hbm.at[src_v.at[pl.ds(c * CHUNK, CHUNK)]], buf, sem
            ).start()

        def wait_gather(buf, sem):
            pltpu.make_async_copy(
                sup_hbm.at[src_v.at[pl.ds(0, CHUNK)]], buf, sem
            ).wait()

        def start_scatter(buf, dv, sem):
            pltpu.async_copy(buf, acc.at[dv], sem, add=True)

        def wait_scatter(buf, dv, sem):
            pltpu.make_async_copy(buf, acc.at[dv], sem).wait()

        start_idx(0, dv0, i0)
        start_gather(0, buf0, g0)
        start_idx(1, dv1, i1)
        start_gather(1, buf1, g1)

        @pl.loop(0, NCHUNKS - 1, step=2)
        def _body(c):
            wait_gather(buf0, g0)
            wait_idx(dv0, i0)
            start_scatter(buf0, dv0, s0)
            wait_gather(buf1, g1)
            wait_idx(dv1, i1)
            start_scatter(buf1, dv1, s1)
            wait_scatter(buf0, dv0, s0)
            start_idx(c + 2, dv0, i0)
            start_gather(c + 2, buf0, g0)
            wait_scatter(buf1, dv1, s1)

            @pl.when(c + 3 < NCHUNKS)
            def _more():
                start_idx(c + 3, dv1, i1)
                start_gather(c + 3, buf1, g1)

        wait_gather(buf0, g0)
        wait_idx(dv0, i0)
        start_scatter(buf0, dv0, s0)
        wait_scatter(buf0, dv0, s0)

        plsc.subcore_barrier()

        pltpu.sync_copy(
            acc.at[pl.ds(base, ROWS_PER_TILE)],
            out_hbm.at[cid].at[pl.ds(base, ROWS_PER_TILE)],
        )

    return kern(support, src2d, dst3d)


def _full_spec():
    return pl.BlockSpec(index_map=lambda i: (0, 0))


def _row_spec():
    return pl.BlockSpec((ROW_BLOCK, F), lambda i: (i, 0))


def _parts_spec():
    return pl.BlockSpec((NC, ROW_BLOCK, F), lambda i: (0, i, 0))


def _mm(a, b):
    return jnp.dot(a, b, preferred_element_type=jnp.float32)


def _tc_head(x, fc1_w, fc1_b, fc2_w, fc2_b, gamma, beta, wn, ws):
    def body(x_ref, w1, b1, w2, b2, g, bt, wn_ref, ws_ref, sup_ref, slf_ref):
        h = jnp.maximum(_mm(x_ref[...], w1[...]) + b1[...], 0.0)
        h = jnp.maximum(_mm(h, w2[...]) + b2[...], 0.0)
        mean = jnp.mean(h, axis=1, keepdims=True)
        var = jnp.sum((h - mean) ** 2, axis=1, keepdims=True) * (1.0 / (F - 1))
        h = g[...] * (h - mean) / (jnp.sqrt(var) + EPS) + bt[...]
        sup_ref[...] = _mm(h, wn_ref[...])
        slf_ref[...] = _mm(h, ws_ref[...])

    return pl.pallas_call(
        body,
        grid=(N_NODES // ROW_BLOCK,),
        in_specs=[_row_spec()] + [_full_spec()] * 8,
        out_specs=[_row_spec(), _row_spec()],
        out_shape=[jax.ShapeDtypeStruct((N_NODES, F), jnp.float32)] * 2,
    )(x, fc1_w, fc1_b, fc2_w, fc2_b, gamma, beta, wn, ws)


def _tc_mid(slf, parts, b, wn, ws):
    def body(slf_ref, p_ref, b_ref, wn_ref, ws_ref, sup_ref, slf_ref_o):
        h = jnp.maximum(slf_ref[...] + p_ref[0] + p_ref[1] + b_ref[...], 0.0)
        sup_ref[...] = _mm(h, wn_ref[...])
        slf_ref_o[...] = _mm(h, ws_ref[...])

    return pl.pallas_call(
        body,
        grid=(N_NODES // ROW_BLOCK,),
        in_specs=[
            _row_spec(),
            _parts_spec(),
            _full_spec(),
            _full_spec(),
            _full_spec(),
        ],
        out_specs=[_row_spec(), _row_spec()],
        out_shape=[jax.ShapeDtypeStruct((N_NODES, F), jnp.float32)] * 2,
    )(slf, parts, b, wn, ws)


def _tc_final(slf, parts, b):
    def body(slf_ref, p_ref, b_ref, out_ref):
        out_ref[...] = jnp.maximum(
            slf_ref[...] + p_ref[0] + p_ref[1] + b_ref[...], 0.0)

    return pl.pallas_call(
        body,
        grid=(N_NODES // ROW_BLOCK,),
        in_specs=[
            _row_spec(),
            _parts_spec(),
            _full_spec(),
        ],
        out_specs=_row_spec(),
        out_shape=jax.ShapeDtypeStruct((N_NODES, F), jnp.float32),
    )(slf, parts, b)


def kernel(x, edge_index, fc1_w, fc1_b, fc2_w, fc2_b,
           gc1_wn, gc1_ws, gc1_b, gc2_wn, gc2_ws, gc2_b,
           gc3_wn, gc3_ws, gc3_b, gc4_wn, gc4_ws, gc4_b,
           ln_gamma, ln_beta):
    ei = edge_index.astype(jnp.int32)
    pad_e = TILE_E - EDGES_PER_TILE
    src2d = jnp.pad(ei[0].reshape(NW, EDGES_PER_TILE), ((0, 0), (0, pad_e)),
                    constant_values=0)
    dst3d = jnp.pad(ei[1].reshape(NW, EDGES_PER_TILE), ((0, 0), (0, pad_e)),
                    constant_values=N_NODES).reshape(NW, NCHUNKS, CHUNK)

    b2 = lambda v: v.reshape(1, F)

    sup, slf = _tc_head(x, fc1_w, b2(fc1_b), fc2_w, b2(fc2_b),
                        b2(ln_gamma), b2(ln_beta), gc1_wn, gc1_ws)

    parts = _segment_sum_sc(sup, src2d, dst3d)
    sup, slf = _tc_mid(slf, parts, b2(gc1_b), gc2_wn, gc2_ws)

    parts = _segment_sum_sc(sup, src2d, dst3d)
    sup, slf = _tc_mid(slf, parts, b2(gc2_b), gc3_wn, gc3_ws)

    parts = _segment_sum_sc(sup, src2d, dst3d)
    sup, slf = _tc_mid(slf, parts, b2(gc3_b), gc4_wn, gc4_ws)

    parts = _segment_sum_sc(sup, src2d, dst3d)
    return _tc_final(slf, parts, b2(gc4_b))


# speedup vs baseline: 1.1281x; 1.1281x over previous
import functools

import jax
import jax.numpy as jnp
from jax import lax
from jax.experimental import pallas as pl
from jax.experimental.pallas import tpu as pltpu
from jax.experimental.pallas import tpu_sc as plsc

N_NODES = 10000
N_EDGES = 320000
F = 128
EPS = 1e-6

NC = 2
NS = 16
NW = NC * NS
EDGES_PER_TILE = N_EDGES // NW
CHUNK = 96
NCHUNKS = -(-EDGES_PER_TILE // CHUNK)
TILE_E = NCHUNKS * CHUNK
N_PAD = 10112
ROWS_PER_TILE = N_PAD // NS

ROW_BLOCK = 2000


def _segment_sum_sc(support, src2d, dst3d):
    mesh = plsc.VectorSubcoreMesh(core_axis_name="c", subcore_axis_name="s")

    @functools.partial(
        pl.kernel,
        out_type=jax.ShapeDtypeStruct((NC, N_PAD, F), jnp.float32),
        mesh=mesh,
        scratch_types=[
            pltpu.VMEM((TILE_E,), jnp.int32),
            pltpu.VMEM((NCHUNKS, CHUNK), jnp.int32),
            pltpu.VMEM((CHUNK, F), jnp.float32),
            pltpu.VMEM((CHUNK, F), jnp.float32),
            pltpu.VMEM_SHARED((N_PAD, F), jnp.float32),
            pltpu.SemaphoreType.DMA,
            pltpu.SemaphoreType.DMA,
            pltpu.SemaphoreType.DMA,
            pltpu.SemaphoreType.DMA,
        ],
    )
    def kern(sup_hbm, src_hbm, dst_hbm, out_hbm,
             src_v, dst_v, buf0, buf1, acc, g0, g1, s0, s1):
        cid = lax.axis_index("c")
        sid = lax.axis_index("s")
        wid = cid * NS + sid

        pltpu.sync_copy(src_hbm.at[wid], src_v)
        pltpu.sync_copy(dst_hbm.at[wid], dst_v)

        @pl.loop(0, CHUNK)
        def _zr(r):
            @pl.loop(0, F, step=16)
            def _zc(c):
                buf0[r, pl.ds(c, 16)] = jnp.zeros((16,), jnp.float32)

        base = sid * ROWS_PER_TILE

        @pl.loop(0, ROWS_PER_TILE - CHUNK, step=CHUNK)
        def _za(r0):
            pltpu.sync_copy(buf0, acc.at[pl.ds(base + r0, CHUNK)])

        pltpu.sync_copy(
            buf0.at[pl.ds(0, ROWS_PER_TILE % CHUNK)],
            acc.at[pl.ds(base + ROWS_PER_TILE - ROWS_PER_TILE % CHUNK,
                         ROWS_PER_TILE % CHUNK)],
        )

        plsc.subcore_barrier()

        def start_gather(c, buf, sem):
            pltpu.make_async_copy(
                sup_hbm.at[src_v.at[pl.ds(c * CHUNK, CHUNK)]], buf, sem
            ).start()

        def wait_gather(buf, sem):
            pltpu.make_async_copy(
                sup_hbm.at[src_v.at[pl.ds(0, CHUNK)]], buf, sem
            ).wait()

        def start_scatter(c, buf, sem):
            pltpu.async_copy(buf, acc.at[dst_v.at[c]], sem, add=True)

        def wait_scatter(c, buf, sem):
            pltpu.make_async_copy(buf, acc.at[dst_v.at[c]], sem).wait()

        start_gather(0, buf0, g0)
        start_gather(1, buf1, g1)

        @pl.loop(0, NCHUNKS - 1, step=2)
        def _body(c):
            wait_gather(buf0, g0)
            start_scatter(c, buf0, s0)
            wait_gather(buf1, g1)
            start_scatter(c + 1, buf1, s1)
            wait_scatter(c, buf0, s0)
            start_gather(c + 2, buf0, g0)
            wait_scatter(c + 1, buf1, s1)

            @pl.when(c + 3 < NCHUNKS)
            def _more():
                start_gather(c + 3, buf1, g1)

        wait_gather(buf0, g0)
        start_scatter(NCHUNKS - 1, buf0, s0)
        wait_scatter(NCHUNKS - 1, buf0, s0)

        plsc.subcore_barrier()

        pltpu.sync_copy(
            acc.at[pl.ds(base, ROWS_PER_TILE)],
            out_hbm.at[cid].at[pl.ds(base, ROWS_PER_TILE)],
        )

    return kern(support, src2d, dst3d)


def _full_spec():
    return pl.BlockSpec(index_map=lambda i: (0, 0))


def _row_spec():
    return pl.BlockSpec((ROW_BLOCK, F), lambda i: (i, 0))


def _parts_spec():
    return pl.BlockSpec((NC, ROW_BLOCK, F), lambda i: (0, i, 0))


def _mm(a, b):
    return jnp.dot(a, b, preferred_element_type=jnp.float32)


def _tc_head(x, fc1_w, fc1_b, fc2_w, fc2_b, gamma, beta, wn, ws):
    def body(x_ref, w1, b1, w2, b2, g, bt, wn_ref, ws_ref, sup_ref, slf_ref):
        h = jnp.maximum(_mm(x_ref[...], w1[...]) + b1[...], 0.0)
        h = jnp.maximum(_mm(h, w2[...]) + b2[...], 0.0)
        mean = jnp.mean(h, axis=1, keepdims=True)
        var = jnp.sum((h - mean) ** 2, axis=1, keepdims=True) * (1.0 / (F - 1))
        h = g[...] * (h - mean) / (jnp.sqrt(var) + EPS) + bt[...]
        sup_ref[...] = _mm(h, wn_ref[...])
        slf_ref[...] = _mm(h, ws_ref[...])

    return pl.pallas_call(
        body,
        grid=(N_NODES // ROW_BLOCK,),
        in_specs=[_row_spec()] + [_full_spec()] * 8,
        out_specs=[_row_spec(), _row_spec()],
        out_shape=[jax.ShapeDtypeStruct((N_NODES, F), jnp.float32)] * 2,
    )(x, fc1_w, fc1_b, fc2_w, fc2_b, gamma, beta, wn, ws)


def _tc_mid(slf, parts, b, wn, ws):
    def body(slf_ref, p_ref, b_ref, wn_ref, ws_ref, sup_ref, slf_ref_o):
        h = jnp.maximum(slf_ref[...] + p_ref[0] + p_ref[1] + b_ref[...], 0.0)
        sup_ref[...] = _mm(h, wn_ref[...])
        slf_ref_o[...] = _mm(h, ws_ref[...])

    return pl.pallas_call(
        body,
        grid=(N_NODES // ROW_BLOCK,),
        in_specs=[
            _row_spec(),
            _parts_spec(),
            _full_spec(),
            _full_spec(),
            _full_spec(),
        ],
        out_specs=[_row_spec(), _row_spec()],
        out_shape=[jax.ShapeDtypeStruct((N_NODES, F), jnp.float32)] * 2,
    )(slf, parts, b, wn, ws)


def _tc_final(slf, parts, b):
    def body(slf_ref, p_ref, b_ref, out_ref):
        out_ref[...] = jnp.maximum(
            slf_ref[...] + p_ref[0] + p_ref[1] + b_ref[...], 0.0)

    return pl.pallas_call(
        body,
        grid=(N_NODES // ROW_BLOCK,),
        in_specs=[
            _row_spec(),
            _parts_spec(),
            _full_spec(),
        ],
        out_specs=_row_spec(),
        out_shape=jax.ShapeDtypeStruct((N_NODES, F), jnp.float32),
    )(slf, parts, b)


def kernel(x, edge_index, fc1_w, fc1_b, fc2_w, fc2_b,
           gc1_wn, gc1_ws, gc1_b, gc2_wn, gc2_ws, gc2_b,
           gc3_wn, gc3_ws, gc3_b, gc4_wn, gc4_ws, gc4_b,
           ln_gamma, ln_beta):
    ei = edge_index.astype(jnp.int32)
    pad_e = TILE_E - EDGES_PER_TILE
    src2d = jnp.pad(ei[0].reshape(NW, EDGES_PER_TILE), ((0, 0), (0, pad_e)),
                    constant_values=0)
    dst3d = jnp.pad(ei[1].reshape(NW, EDGES_PER_TILE), ((0, 0), (0, pad_e)),
                    constant_values=N_NODES).reshape(NW, NCHUNKS, CHUNK)

    b2 = lambda v: v.reshape(1, F)

    sup, slf = _tc_head(x, fc1_w, b2(fc1_b), fc2_w, b2(fc2_b),
                        b2(ln_gamma), b2(ln_beta), gc1_wn, gc1_ws)

    parts = _segment_sum_sc(sup, src2d, dst3d)
    sup, slf = _tc_mid(slf, parts, b2(gc1_b), gc2_wn, gc2_ws)

    parts = _segment_sum_sc(sup, src2d, dst3d)
    sup, slf = _tc_mid(slf, parts, b2(gc2_b), gc3_wn, gc3_ws)

    parts = _segment_sum_sc(sup, src2d, dst3d)
    sup, slf = _tc_mid(slf, parts, b2(gc3_b), gc4_wn, gc4_ws)

    parts = _segment_sum_sc(sup, src2d, dst3d)
    return _tc_final(slf, parts, b2(gc4_b))


# speedup vs baseline: 1.2982x; 1.1507x over previous
import functools

import jax
import jax.numpy as jnp
from jax import lax
from jax.experimental import pallas as pl
from jax.experimental.pallas import tpu as pltpu
from jax.experimental.pallas import tpu_sc as plsc

N_NODES = 10000
N_EDGES = 320000
F = 128
EPS = 1e-6

NC = 2
NS = 16
NW = NC * NS
EDGES_PER_TILE = N_EDGES // NW
CHUNK = 96
NCHUNKS = -(-EDGES_PER_TILE // CHUNK)
TILE_E = NCHUNKS * CHUNK
N_PAD = 10112
ROWS_PER_TILE = N_PAD // NS

ROW_BLOCK = 2000


def _segment_sum_sc(support, src2d, dst3d):
    mesh = plsc.VectorSubcoreMesh(core_axis_name="c", subcore_axis_name="s")

    @functools.partial(
        pl.kernel,
        out_type=jax.ShapeDtypeStruct((NC, N_PAD, F), jnp.float32),
        mesh=mesh,
        scratch_types=[
            pltpu.VMEM((TILE_E,), jnp.int32),
            pltpu.VMEM((NCHUNKS, CHUNK), jnp.int32),
            pltpu.VMEM((CHUNK, F), jnp.float32),
            pltpu.VMEM((CHUNK, F), jnp.float32),
            pltpu.VMEM_SHARED((N_PAD, F), jnp.float32),
            pltpu.SemaphoreType.DMA,
            pltpu.SemaphoreType.DMA,
        ],
    )
    def kern(sup_hbm, src_hbm, dst_hbm, out_hbm,
             src_v, dst_v, buf0, buf1, acc, g0, g1):
        cid = lax.axis_index("c")
        sid = lax.axis_index("s")
        wid = cid * NS + sid

        pltpu.sync_copy(src_hbm.at[wid], src_v)
        pltpu.sync_copy(dst_hbm.at[wid], dst_v)

        @pl.loop(0, CHUNK)
        def _zr(r):
            @pl.loop(0, F, step=16)
            def _zc(c):
                buf0[r, pl.ds(c, 16)] = jnp.zeros((16,), jnp.float32)

        base = sid * ROWS_PER_TILE

        @pl.loop(0, ROWS_PER_TILE - CHUNK, step=CHUNK)
        def _za(r0):
            pltpu.sync_copy(buf0, acc.at[pl.ds(base + r0, CHUNK)])

        pltpu.sync_copy(
            buf0.at[pl.ds(0, ROWS_PER_TILE % CHUNK)],
            acc.at[pl.ds(base + ROWS_PER_TILE - ROWS_PER_TILE % CHUNK,
                         ROWS_PER_TILE % CHUNK)],
        )

        plsc.subcore_barrier()

        def start_gather(c, buf, sem):
            pltpu.make_async_copy(
                sup_hbm.at[src_v.at[pl.ds(c * CHUNK, CHUNK)]], buf, sem
            ).start()

        def wait_gather(buf, sem):
            pltpu.make_async_copy(
                sup_hbm.at[src_v.at[pl.ds(0, CHUNK)]], buf, sem
            ).wait()

        def scatter_add(c, buf):
            pltpu.sync_copy(buf, acc.at[dst_v.at[c]], add=True)

        start_gather(0, buf0, g0)

        @pl.loop(0, NCHUNKS - 1, step=2)
        def _body(c):
            start_gather(c + 1, buf1, g1)
            wait_gather(buf0, g0)
            scatter_add(c, buf0)
            start_gather(c + 2, buf0, g0)
            wait_gather(buf1, g1)
            scatter_add(c + 1, buf1)

        wait_gather(buf0, g0)
        scatter_add(NCHUNKS - 1, buf0)

        plsc.subcore_barrier()

        pltpu.sync_copy(
            acc.at[pl.ds(base, ROWS_PER_TILE)],
            out_hbm.at[cid].at[pl.ds(base, ROWS_PER_TILE)],
        )

    return kern(support, src2d, dst3d)


def _full_spec():
    return pl.BlockSpec(index_map=lambda i: (0, 0))


def _row_spec():
    return pl.BlockSpec((ROW_BLOCK, F), lambda i: (i, 0))


def _parts_spec():
    return pl.BlockSpec((NC, ROW_BLOCK, F), lambda i: (0, i, 0))


def _mm(a, b):
    return jnp.dot(a, b, preferred_element_type=jnp.float32)


def _tc_head(x, fc1_w, fc1_b, fc2_w, fc2_b, gamma, beta, wn, ws):
    def body(x_ref, w1, b1, w2, b2, g, bt, wn_ref, ws_ref, sup_ref, slf_ref):
        h = jnp.maximum(_mm(x_ref[...], w1[...]) + b1[...], 0.0)
        h = jnp.maximum(_mm(h, w2[...]) + b2[...], 0.0)
        mean = jnp.mean(h, axis=1, keepdims=True)
        var = jnp.sum((h - mean) ** 2, axis=1, keepdims=True) * (1.0 / (F - 1))
        h = g[...] * (h - mean) / (jnp.sqrt(var) + EPS) + bt[...]
        sup_ref[...] = _mm(h, wn_ref[...])
        slf_ref[...] = _mm(h, ws_ref[...])

    return pl.pallas_call(
        body,
        grid=(N_NODES // ROW_BLOCK,),
        in_specs=[_row_spec()] + [_full_spec()] * 8,
        out_specs=[_row_spec(), _row_spec()],
        out_shape=[jax.ShapeDtypeStruct((N_NODES, F), jnp.float32)] * 2,
    )(x, fc1_w, fc1_b, fc2_w, fc2_b, gamma, beta, wn, ws)


def _tc_mid(slf, parts, b, wn, ws):
    def body(slf_ref, p_ref, b_ref, wn_ref, ws_ref, sup_ref, slf_ref_o):
        h = jnp.maximum(slf_ref[...] + p_ref[0] + p_ref[1] + b_ref[...], 0.0)
        sup_ref[...] = _mm(h, wn_ref[...])
        slf_ref_o[...] = _mm(h, ws_ref[...])

    return pl.pallas_call(
        body,
        grid=(N_NODES // ROW_BLOCK,),
        in_specs=[
            _row_spec(),
            _parts_spec(),
            _full_spec(),
            _full_spec(),
            _full_spec(),
        ],
        out_specs=[_row_spec(), _row_spec()],
        out_shape=[jax.ShapeDtypeStruct((N_NODES, F), jnp.float32)] * 2,
    )(slf, parts, b, wn, ws)


def _tc_final(slf, parts, b):
    def body(slf_ref, p_ref, b_ref, out_ref):
        out_ref[...] = jnp.maximum(
            slf_ref[...] + p_ref[0] + p_ref[1] + b_ref[...], 0.0)

    return pl.pallas_call(
        body,
        grid=(N_NODES // ROW_BLOCK,),
        in_specs=[
            _row_spec(),
            _parts_spec(),
            _full_spec(),
        ],
        out_specs=_row_spec(),
        out_shape=jax.ShapeDtypeStruct((N_NODES, F), jnp.float32),
    )(slf, parts, b)


def kernel(x, edge_index, fc1_w, fc1_b, fc2_w, fc2_b,
           gc1_wn, gc1_ws, gc1_b, gc2_wn, gc2_ws, gc2_b,
           gc3_wn, gc3_ws, gc3_b, gc4_wn, gc4_ws, gc4_b,
           ln_gamma, ln_beta):
    ei = edge_index.astype(jnp.int32)
    pad_e = TILE_E - EDGES_PER_TILE
    src2d = jnp.pad(ei[0].reshape(NW, EDGES_PER_TILE), ((0, 0), (0, pad_e)),
                    constant_values=0)
    dst3d = jnp.pad(ei[1].reshape(NW, EDGES_PER_TILE), ((0, 0), (0, pad_e)),
                    constant_values=N_NODES).reshape(NW, NCHUNKS, CHUNK)

    b2 = lambda v: v.reshape(1, F)

    sup, slf = _tc_head(x, fc1_w, b2(fc1_b), fc2_w, b2(fc2_b),
                        b2(ln_gamma), b2(ln_beta), gc1_wn, gc1_ws)

    parts = _segment_sum_sc(sup, src2d, dst3d)
    sup, slf = _tc_mid(slf, parts, b2(gc1_b), gc2_wn, gc2_ws)

    parts = _segment_sum_sc(sup, src2d, dst3d)
    sup, slf = _tc_mid(slf, parts, b2(gc2_b), gc3_wn, gc3_ws)

    parts = _segment_sum_sc(sup, src2d, dst3d)
    sup, slf = _tc_mid(slf, parts, b2(gc3_b), gc4_wn, gc4_ws)

    parts = _segment_sum_sc(sup, src2d, dst3d)
    return _tc_final(slf, parts, b2(gc4_b))


# speedup vs baseline: 1.3092x; 1.0085x over previous
import functools

import jax
import jax.numpy as jnp
from jax import lax
from jax.experimental import pallas as pl
from jax.experimental.pallas import tpu as pltpu
from jax.experimental.pallas import tpu_sc as plsc

N_NODES = 10000
N_EDGES = 320000
F = 128
EPS = 1e-6

NC = 2
NS = 16
NW = NC * NS
EDGES_PER_TILE = N_EDGES // NW
CHUNK = 96
NCHUNKS = -(-EDGES_PER_TILE // CHUNK)
TILE_E = NCHUNKS * CHUNK
N_PAD = 10112
ROWS_PER_TILE = N_PAD // NS

ROW_BLOCK = 2000


def _segment_sum_sc(support, src2d, dst3d):
    mesh = plsc.VectorSubcoreMesh(core_axis_name="c", subcore_axis_name="s")

    @functools.partial(
        pl.kernel,
        out_type=jax.ShapeDtypeStruct((NC, N_PAD, F), jnp.float32),
        mesh=mesh,
        scratch_types=[
            pltpu.VMEM((TILE_E,), jnp.int32),
            pltpu.VMEM((NCHUNKS, CHUNK), jnp.int32),
            pltpu.VMEM((CHUNK, F), jnp.float32),
            pltpu.VMEM((CHUNK, F), jnp.float32),
            pltpu.VMEM_SHARED((N_PAD, F), jnp.float32),
            pltpu.SemaphoreType.DMA,
            pltpu.SemaphoreType.DMA,
            pltpu.SemaphoreType.DMA,
            pltpu.SemaphoreType.DMA,
        ],
    )
    def kern(sup_hbm, src_hbm, dst_hbm, out_hbm,
             src_v, dst_v, buf0, buf1, acc, g0, g1, g2, g3):
        cid = lax.axis_index("c")
        sid = lax.axis_index("s")
        wid = cid * NS + sid

        pltpu.sync_copy(src_hbm.at[wid], src_v)
        pltpu.sync_copy(dst_hbm.at[wid], dst_v)

        @pl.loop(0, CHUNK)
        def _zr(r):
            @pl.loop(0, F, step=16)
            def _zc(c):
                buf0[r, pl.ds(c, 16)] = jnp.zeros((16,), jnp.float32)

        base = sid * ROWS_PER_TILE

        @pl.loop(0, ROWS_PER_TILE - CHUNK, step=CHUNK)
        def _za(r0):
            pltpu.sync_copy(buf0, acc.at[pl.ds(base + r0, CHUNK)])

        pltpu.sync_copy(
            buf0.at[pl.ds(0, ROWS_PER_TILE % CHUNK)],
            acc.at[pl.ds(base + ROWS_PER_TILE - ROWS_PER_TILE % CHUNK,
                         ROWS_PER_TILE % CHUNK)],
        )

        plsc.subcore_barrier()

        H = CHUNK // 2

        def start_gather(c, buf, semA, semB):
            pltpu.make_async_copy(
                sup_hbm.at[src_v.at[pl.ds(c * CHUNK, H)]],
                buf.at[pl.ds(0, H)], semA,
            ).start()
            pltpu.make_async_copy(
                sup_hbm.at[src_v.at[pl.ds(c * CHUNK + H, H)]],
                buf.at[pl.ds(H, H)], semB,
            ).start()

        def wait_gather(buf, semA, semB):
            pltpu.make_async_copy(
                sup_hbm.at[src_v.at[pl.ds(0, H)]], buf.at[pl.ds(0, H)], semA
            ).wait()
            pltpu.make_async_copy(
                sup_hbm.at[src_v.at[pl.ds(0, H)]], buf.at[pl.ds(H, H)], semB
            ).wait()

        def scatter_add(c, buf):
            pltpu.sync_copy(buf, acc.at[dst_v.at[c]], add=True)

        start_gather(0, buf0, g0, g1)

        @pl.loop(0, NCHUNKS - 1, step=2)
        def _body(c):
            start_gather(c + 1, buf1, g2, g3)
            wait_gather(buf0, g0, g1)
            scatter_add(c, buf0)
            start_gather(c + 2, buf0, g0, g1)
            wait_gather(buf1, g2, g3)
            scatter_add(c + 1, buf1)

        wait_gather(buf0, g0, g1)
        scatter_add(NCHUNKS - 1, buf0)

        plsc.subcore_barrier()

        pltpu.sync_copy(
            acc.at[pl.ds(base, ROWS_PER_TILE)],
            out_hbm.at[cid].at[pl.ds(base, ROWS_PER_TILE)],
        )

    return kern(support, src2d, dst3d)


def _full_spec():
    return pl.BlockSpec(index_map=lambda i: (0, 0))


def _row_spec():
    return pl.BlockSpec((ROW_BLOCK, F), lambda i: (i, 0))


def _parts_spec():
    return pl.BlockSpec((NC, ROW_BLOCK, F), lambda i: (0, i, 0))


def _mm(a, b):
    return jnp.dot(a, b, preferred_element_type=jnp.float32)


def _tc_head(x, fc1_w, fc1_b, fc2_w, fc2_b, gamma, beta, wn, ws):
    def body(x_ref, w1, b1, w2, b2, g, bt, wn_ref, ws_ref, sup_ref, slf_ref):
        h = jnp.maximum(_mm(x_ref[...], w1[...]) + b1[...], 0.0)
        h = jnp.maximum(_mm(h, w2[...]) + b2[...], 0.0)
        mean = jnp.mean(h, axis=1, keepdims=True)
        var = jnp.sum((h - mean) ** 2, axis=1, keepdims=True) * (1.0 / (F - 1))
        h = g[...] * (h - mean) / (jnp.sqrt(var) + EPS) + bt[...]
        sup_ref[...] = _mm(h, wn_ref[...])
        slf_ref[...] = _mm(h, ws_ref[...])

    return pl.pallas_call(
        body,
        grid=(N_NODES // ROW_BLOCK,),
        in_specs=[_row_spec()] + [_full_spec()] * 8,
        out_specs=[_row_spec(), _row_spec()],
        out_shape=[jax.ShapeDtypeStruct((N_NODES, F), jnp.float32)] * 2,
    )(x, fc1_w, fc1_b, fc2_w, fc2_b, gamma, beta, wn, ws)


def _tc_mid(slf, parts, b, wn, ws):
    def body(slf_ref, p_ref, b_ref, wn_ref, ws_ref, sup_ref, slf_ref_o):
        h = jnp.maximum(slf_ref[...] + p_ref[0] + p_ref[1] + b_ref[...], 0.0)
        sup_ref[...] = _mm(h, wn_ref[...])
        slf_ref_o[...] = _mm(h, ws_ref[...])

    return pl.pallas_call(
        body,
        grid=(N_NODES // ROW_BLOCK,),
        in_specs=[
            _row_spec(),
            _parts_spec(),
            _full_spec(),
            _full_spec(),
            _full_spec(),
        ],
        out_specs=[_row_spec(), _row_spec()],
        out_shape=[jax.ShapeDtypeStruct((N_NODES, F), jnp.float32)] * 2,
    )(slf, parts, b, wn, ws)


def _tc_final(slf, parts, b):
    def body(slf_ref, p_ref, b_ref, out_ref):
        out_ref[...] = jnp.maximum(
            slf_ref[...] + p_ref[0] + p_ref[1] + b_ref[...], 0.0)

    return pl.pallas_call(
        body,
        grid=(N_NODES // ROW_BLOCK,),
        in_specs=[
            _row_spec(),
            _parts_spec(),
            _full_spec(),
        ],
        out_specs=_row_spec(),
        out_shape=jax.ShapeDtypeStruct((N_NODES, F), jnp.float32),
    )(slf, parts, b)


def kernel(x, edge_index, fc1_w, fc1_b, fc2_w, fc2_b,
           gc1_wn, gc1_ws, gc1_b, gc2_wn, gc2_ws, gc2_b,
           gc3_wn, gc3_ws, gc3_b, gc4_wn, gc4_ws, gc4_b,
           ln_gamma, ln_beta):
    ei = edge_index.astype(jnp.int32)
    pad_e = TILE_E - EDGES_PER_TILE
    src2d = jnp.pad(ei[0].reshape(NW, EDGES_PER_TILE), ((0, 0), (0, pad_e)),
                    constant_values=0)
    dst3d = jnp.pad(ei[1].reshape(NW, EDGES_PER_TILE), ((0, 0), (0, pad_e)),
                    constant_values=N_NODES).reshape(NW, NCHUNKS, CHUNK)

    b2 = lambda v: v.reshape(1, F)

    sup, slf = _tc_head(x, fc1_w, b2(fc1_b), fc2_w, b2(fc2_b),
                        b2(ln_gamma), b2(ln_beta), gc1_wn, gc1_ws)

    parts = _segment_sum_sc(sup, src2d, dst3d)
    sup, slf = _tc_mid(slf, parts, b2(gc1_b), gc2_wn, gc2_ws)

    parts = _segment_sum_sc(sup, src2d, dst3d)
    sup, slf = _tc_mid(slf, parts, b2(gc2_b), gc3_wn, gc3_ws)

    parts = _segment_sum_sc(sup, src2d, dst3d)
    sup, slf = _tc_mid(slf, parts, b2(gc3_b), gc4_wn, gc4_ws)

    parts = _segment_sum_sc(sup, src2d, dst3d)
    return _tc_final(slf, parts, b2(gc4_b))


# speedup vs baseline: 1.3246x; 1.0118x over previous
import functools

import jax
import jax.numpy as jnp
from jax import lax
from jax.experimental import pallas as pl
from jax.experimental.pallas import tpu as pltpu
from jax.experimental.pallas import tpu_sc as plsc

N_NODES = 10000
N_EDGES = 320000
F = 128
EPS = 1e-6

NC = 2
NS = 16
NW = NC * NS
EDGES_PER_TILE = N_EDGES // NW
CHUNK = 96
NCHUNKS = -(-EDGES_PER_TILE // CHUNK)
TILE_E = NCHUNKS * CHUNK
N_PAD = 10112
ROWS_PER_TILE = N_PAD // NS

ROW_BLOCK = 2000


def _segment_sum_sc(support, src2d, dst3d):
    mesh = plsc.VectorSubcoreMesh(core_axis_name="c", subcore_axis_name="s")

    @functools.partial(
        pl.kernel,
        out_type=jax.ShapeDtypeStruct((NC, N_PAD, F), jnp.float32),
        mesh=mesh,
        scratch_types=[
            pltpu.VMEM((TILE_E,), jnp.int32),
            pltpu.VMEM((NCHUNKS, CHUNK), jnp.int32),
            pltpu.VMEM((CHUNK, F), jnp.float32),
            pltpu.VMEM((CHUNK, F), jnp.float32),
            pltpu.VMEM_SHARED((N_PAD, F), jnp.float32),
            pltpu.SemaphoreType.DMA,
            pltpu.SemaphoreType.DMA,
            pltpu.SemaphoreType.DMA,
            pltpu.SemaphoreType.DMA,
        ],
    )
    def kern(sup_hbm, src_hbm, dst_hbm, out_hbm,
             src_v, dst_v, buf0, buf1, acc, g0, g1, g2, g3):
        cid = lax.axis_index("c")
        sid = lax.axis_index("s")
        wid = cid * NS + sid

        base = sid * ROWS_PER_TILE
        TAIL = ROWS_PER_TILE % CHUNK

        pltpu.sync_copy(src_hbm.at[wid], src_v)
        pltpu.make_async_copy(dst_hbm.at[wid], dst_v, g3).start()

        H = CHUNK // 2

        def start_gather(c, buf, semA, semB):
            pltpu.make_async_copy(
                sup_hbm.at[src_v.at[pl.ds(c * CHUNK, H)]],
                buf.at[pl.ds(0, H)], semA,
            ).start()
            pltpu.make_async_copy(
                sup_hbm.at[src_v.at[pl.ds(c * CHUNK + H, H)]],
                buf.at[pl.ds(H, H)], semB,
            ).start()

        def wait_gather(buf, semA, semB):
            pltpu.make_async_copy(
                sup_hbm.at[src_v.at[pl.ds(0, H)]], buf.at[pl.ds(0, H)], semA
            ).wait()
            pltpu.make_async_copy(
                sup_hbm.at[src_v.at[pl.ds(0, H)]], buf.at[pl.ds(H, H)], semB
            ).wait()

        def scatter_add(c, buf):
            pltpu.sync_copy(buf, acc.at[dst_v.at[c]], add=True)

        start_gather(0, buf1, g0, g1)

        @pl.loop(0, CHUNK)
        def _zr(r):
            @pl.loop(0, F, step=16)
            def _zc(c):
                buf0[r, pl.ds(c, 16)] = jnp.zeros((16,), jnp.float32)

        @pl.loop(0, ROWS_PER_TILE - CHUNK, step=CHUNK)
        def _za(r0):
            pltpu.make_async_copy(buf0, acc.at[pl.ds(base + r0, CHUNK)],
                                  g2).start()

        pltpu.make_async_copy(
            buf0.at[pl.ds(0, TAIL)],
            acc.at[pl.ds(base + ROWS_PER_TILE - TAIL, TAIL)], g2,
        ).start()

        @pl.loop(0, ROWS_PER_TILE - CHUNK, step=CHUNK)
        def _zw(r0):
            pltpu.make_async_copy(buf0, acc.at[pl.ds(base + r0, CHUNK)],
                                  g2).wait()

        pltpu.make_async_copy(
            buf0.at[pl.ds(0, TAIL)],
            acc.at[pl.ds(base + ROWS_PER_TILE - TAIL, TAIL)], g2,
        ).wait()
        pltpu.make_async_copy(dst_hbm.at[wid], dst_v, g3).wait()

        plsc.subcore_barrier()

        @pl.loop(0, NCHUNKS - 1, step=2)
        def _body(c):
            start_gather(c + 1, buf0, g2, g3)
            wait_gather(buf1, g0, g1)
            scatter_add(c, buf1)
            start_gather(c + 2, buf1, g0, g1)
            wait_gather(buf0, g2, g3)
            scatter_add(c + 1, buf0)

        wait_gather(buf1, g0, g1)
        scatter_add(NCHUNKS - 1, buf1)

        plsc.subcore_barrier()

        pltpu.sync_copy(
            acc.at[pl.ds(base, ROWS_PER_TILE)],
            out_hbm.at[cid].at[pl.ds(base, ROWS_PER_TILE)],
        )

    return kern(support, src2d, dst3d)


def _full_spec():
    return pl.BlockSpec(index_map=lambda i: (0, 0))


def _row_spec():
    return pl.BlockSpec((ROW_BLOCK, F), lambda i: (i, 0))


def _parts_spec():
    return pl.BlockSpec((NC, ROW_BLOCK, F), lambda i: (0, i, 0))


def _mm(a, b):
    return jnp.dot(a, b, preferred_element_type=jnp.float32)


def _tc_head(x, fc1_w, fc1_b, fc2_w, fc2_b, gamma, beta, wn, ws):
    def body(x_ref, w1, b1, w2, b2, g, bt, wn_ref, ws_ref, sup_ref, slf_ref):
        h = jnp.maximum(_mm(x_ref[...], w1[...]) + b1[...], 0.0)
        h = jnp.maximum(_mm(h, w2[...]) + b2[...], 0.0)
        mean = jnp.mean(h, axis=1, keepdims=True)
        var = jnp.sum((h - mean) ** 2, axis=1, keepdims=True) * (1.0 / (F - 1))
        h = g[...] * (h - mean) / (jnp.sqrt(var) + EPS) + bt[...]
        sup_ref[...] = _mm(h, wn_ref[...])
        slf_ref[...] = _mm(h, ws_ref[...])

    return pl.pallas_call(
        body,
        grid=(N_NODES // ROW_BLOCK,),
        in_specs=[_row_spec()] + [_full_spec()] * 8,
        out_specs=[_row_spec(), _row_spec()],
        out_shape=[jax.ShapeDtypeStruct((N_NODES, F), jnp.float32)] * 2,
    )(x, fc1_w, fc1_b, fc2_w, fc2_b, gamma, beta, wn, ws)


def _tc_mid(slf, parts, b, wn, ws):
    def body(slf_ref, p_ref, b_ref, wn_ref, ws_ref, sup_ref, slf_ref_o):
        h = jnp.maximum(slf_ref[...] + p_ref[0] + p_ref[1] + b_ref[...], 0.0)
        sup_ref[...] = _mm(h, wn_ref[...])
        slf_ref_o[...] = _mm(h, ws_ref[...])

    return pl.pallas_call(
        body,
        grid=(N_NODES // ROW_BLOCK,),
        in_specs=[
            _row_spec(),
            _parts_spec(),
            _full_spec(),
            _full_spec(),
            _full_spec(),
        ],
        out_specs=[_row_spec(), _row_spec()],
        out_shape=[jax.ShapeDtypeStruct((N_NODES, F), jnp.float32)] * 2,
    )(slf, parts, b, wn, ws)


def _tc_final(slf, parts, b):
    def body(slf_ref, p_ref, b_ref, out_ref):
        out_ref[...] = jnp.maximum(
            slf_ref[...] + p_ref[0] + p_ref[1] + b_ref[...], 0.0)

    return pl.pallas_call(
        body,
        grid=(N_NODES // ROW_BLOCK,),
        in_specs=[
            _row_spec(),
            _parts_spec(),
            _full_spec(),
        ],
        out_specs=_row_spec(),
        out_shape=jax.ShapeDtypeStruct((N_NODES, F), jnp.float32),
    )(slf, parts, b)


def kernel(x, edge_index, fc1_w, fc1_b, fc2_w, fc2_b,
           gc1_wn, gc1_ws, gc1_b, gc2_wn, gc2_ws, gc2_b,
           gc3_wn, gc3_ws, gc3_b, gc4_wn, gc4_ws, gc4_b,
           ln_gamma, ln_beta):
    ei = edge_index.astype(jnp.int32)
    pad_e = TILE_E - EDGES_PER_TILE
    src2d = jnp.pad(ei[0].reshape(NW, EDGES_PER_TILE), ((0, 0), (0, pad_e)),
                    constant_values=0)
    dst3d = jnp.pad(ei[1].reshape(NW, EDGES_PER_TILE), ((0, 0), (0, pad_e)),
                    constant_values=N_NODES).reshape(NW, NCHUNKS, CHUNK)

    b2 = lambda v: v.reshape(1, F)

    sup, slf = _tc_head(x, fc1_w, b2(fc1_b), fc2_w, b2(fc2_b),
                        b2(ln_gamma), b2(ln_beta), gc1_wn, gc1_ws)

    parts = _segment_sum_sc(sup, src2d, dst3d)
    sup, slf = _tc_mid(slf, parts, b2(gc1_b), gc2_wn, gc2_ws)

    parts = _segment_sum_sc(sup, src2d, dst3d)
    sup, slf = _tc_mid(slf, parts, b2(gc2_b), gc3_wn, gc3_ws)

    parts = _segment_sum_sc(sup, src2d, dst3d)
    sup, slf = _tc_mid(slf, parts, b2(gc3_b), gc4_wn, gc4_ws)

    parts = _segment_sum_sc(sup, src2d, dst3d)
    return _tc_final(slf, parts, b2(gc4_b))
